# baseline (speedup 1.0000x reference)
import functools

import jax
import jax.numpy as jnp
from jax import lax
from jax.experimental import pallas as pl
from jax.experimental.pallas import tpu as pltpu
from jax.experimental.pallas import tpu_sc as plsc

N = 100000
E = 6400000
ROW = 128
NROWS = E // ROW
NC = 2
NS = 16
NW = NC * NS
CH = 16
CE = CH * ROW
NCHUNKS = NROWS // CH

A_FULL = N // ROW
A_TAIL_START = N - ROW


def _sc_kernel(e1d, n_id_src, n_id_dst,
               scale_src, bias_src, std_src,
               scale_dst, bias_dst, std_dst,
               out,
               idx, vbuf,
               f0, f1, f2, f3, f4, f5,
               ld0, ld1, g0, g1, st0, st1):
    c = lax.axis_index("c")
    s = lax.axis_index("s")

    fused = (f0, f1, f2, f3, f4, f5)
    sem_ld = (ld0, ld1)
    sem_g = (g0, g1)
    sem_st = (st0, st1)
    src_params = (scale_src, bias_src, std_src)
    dst_params = (scale_dst, bias_dst, std_dst)

    n_a = jnp.where(s < (A_FULL % NS), A_FULL // NS + 1, A_FULL // NS)

    def a_fire_loads(t, slot):
        start = (s + t * NS) * ROW
        pltpu.async_copy(n_id_src.at[pl.ds(start, ROW)], idx.at[slot, 0, 0], sem_ld[slot])
        pltpu.async_copy(n_id_dst.at[pl.ds(start, ROW)], idx.at[slot, 1, 0], sem_ld[slot])

    def a_wait_loads(slot):
        pltpu.make_async_copy(n_id_src.at[pl.ds(0, ROW)], idx.at[slot, 0, 0], sem_ld[slot]).wait()
        pltpu.make_async_copy(n_id_dst.at[pl.ds(0, ROW)], idx.at[slot, 1, 0], sem_ld[slot]).wait()

    def a_wait_stores(slot):
        for k in range(6):
            pltpu.make_async_copy(vbuf.at[slot, k, pl.ds(0, ROW)], fused[k].at[pl.ds(0, ROW)],
                                  sem_st[slot]).wait()

    def a_chunk_body(t, slot):
        start = (s + t * NS) * ROW

        @pl.when(t >= 2)
        def _():
            a_wait_stores(slot)

        @pl.when(t + 1 < n_a)
        def _():
            a_fire_loads(t + 1, 1 - slot)

        a_wait_loads(slot)
        for k in range(3):
            pltpu.async_copy(src_params[k].at[idx.at[slot, 0, 0]],
                             vbuf.at[slot, k, pl.ds(0, ROW)], sem_g[slot])
            pltpu.async_copy(dst_params[k].at[idx.at[slot, 1, 0]],
                             vbuf.at[slot, 3 + k, pl.ds(0, ROW)], sem_g[slot])
        for k in range(6):
            pltpu.make_async_copy(src_params[0].at[pl.ds(0, ROW)],
                                  vbuf.at[slot, k, pl.ds(0, ROW)], sem_g[slot]).wait()
        for k in range(6):
            pltpu.async_copy(vbuf.at[slot, k, pl.ds(0, ROW)], fused[k].at[pl.ds(start, ROW)],
                             sem_st[slot])

    a_fire_loads(0, 0)

    def body_a(p, carry):
        a_chunk_body(2 * p, 0)

        @pl.when(2 * p + 1 < n_a)
        def _():
            a_chunk_body(2 * p + 1, 1)

        return carry

    lax.fori_loop(0, (n_a + 1) // 2, body_a, 0)
    a_wait_stores(0)
    a_wait_stores(1)

    @pl.when(s == NS - 1)
    def _():
        tail_loads = [
            pltpu.async_copy(n_id_src.at[pl.ds(A_TAIL_START, ROW)], idx.at[0, 0, 0], ld0),
            pltpu.async_copy(n_id_dst.at[pl.ds(A_TAIL_START, ROW)], idx.at[0, 1, 0], ld0)]
        for cp in tail_loads:
            cp.wait()
        gs = []
        for k in range(3):
            gs.append(pltpu.async_copy(src_params[k].at[idx.at[0, 0, 0]],
                                       vbuf.at[0, k, pl.ds(0, ROW)], g0))
            gs.append(pltpu.async_copy(dst_params[k].at[idx.at[0, 1, 0]],
                                       vbuf.at[0, 3 + k, pl.ds(0, ROW)], g0))
        for g in gs:
            g.wait()
        ws = [pltpu.async_copy(vbuf.at[0, k, pl.ds(0, ROW)], fused[k].at[pl.ds(A_TAIL_START, ROW)], st0)
              for k in range(6)]
        for cp in ws:
            cp.wait()

    plsc.subcore_barrier()

    w = s * NC + c
    n_t = (NCHUNKS - w + NW - 1) // NW

    def fire_loads(t, slot):
        g = w + NW * t
        for j in range(CH):
            pltpu.async_copy(e1d.at[pl.ds((g * CH + j) * ROW, ROW)],
                             idx.at[slot, 0, j], sem_ld[slot])
            pltpu.async_copy(e1d.at[pl.ds(E + (g * CH + j) * ROW, ROW)],
                             idx.at[slot, 1, j], sem_ld[slot])

    def wait_loads(slot):
        for j in range(CH):
            pltpu.make_async_copy(e1d.at[pl.ds(0, ROW)], idx.at[slot, 0, j], sem_ld[slot]).wait()
            pltpu.make_async_copy(e1d.at[pl.ds(0, ROW)], idx.at[slot, 1, j], sem_ld[slot]).wait()

    def wait_block(sem, slot):
        for k in range(6):
            pltpu.make_async_copy(out.at[pl.ds(0, CE)], vbuf.at[slot, k], sem[slot]).wait()

    def chunk_body(t, slot):
        g = w + NW * t

        @pl.when(t >= 2)
        def _():
            wait_block(sem_st, slot)

        @pl.when(t + 1 < n_t)
        def _():
            fire_loads(t + 1, 1 - slot)

        wait_loads(slot)
        for j in range(CH):
            for k in range(3):
                pltpu.async_copy(fused[k].at[idx.at[slot, 0, j]],
                                 vbuf.at[slot, k, pl.ds(j * ROW, ROW)], sem_g[slot])
                pltpu.async_copy(fused[3 + k].at[idx.at[slot, 1, j]],
                                 vbuf.at[slot, 3 + k, pl.ds(j * ROW, ROW)], sem_g[slot])
        wait_block(sem_g, slot)
        for k in range(6):
            pltpu.async_copy(vbuf.at[slot, k], out.at[pl.ds(k * E + g * CE, CE)], sem_st[slot])

    fire_loads(0, 0)

    def body_b(p, carry):
        chunk_body(2 * p, 0)

        @pl.when(2 * p + 1 < n_t)
        def _():
            chunk_body(2 * p + 1, 1)

        return carry

    lax.fori_loop(0, (n_t + 1) // 2, body_b, 0)

    wait_block(sem_st, 0)
    wait_block(sem_st, 1)


@jax.jit
def _run(e1d, n_id_src, n_id_dst,
         scale_src, bias_src, std_src, scale_dst, bias_dst, std_dst):
    mesh = plsc.VectorSubcoreMesh(core_axis_name="c", subcore_axis_name="s")
    kfn = functools.partial(
        pl.kernel,
        mesh=mesh,
        out_type=jax.ShapeDtypeStruct((6 * E,), jnp.float32),
        scratch_types=[
            pltpu.VMEM((2, 2, CH, ROW), jnp.int32),
            pltpu.VMEM((2, 6, CE), jnp.float32),
        ] + [pltpu.VMEM_SHARED((N,), jnp.float32) for _ in range(6)]
        + [pltpu.SemaphoreType.DMA for _ in range(6)],
        compiler_params=pltpu.CompilerParams(use_tc_tiling_on_sc=False),
    )(_sc_kernel)
    return kfn(e1d, n_id_src, n_id_dst,
               scale_src, bias_src, std_src,
               scale_dst, bias_dst, std_dst)


def kernel(edge_index, n_id_src, n_id_dst, scale_src, bias_src, std_src,
           scale_dst, bias_dst, std_dst):
    e1d = edge_index.reshape(2 * E)
    out = _run(e1d, n_id_src, n_id_dst,
               scale_src, bias_src, std_src,
               scale_dst, bias_dst, std_dst)
    return out.reshape(6, E)


# speedup vs baseline: 6.7152x; 6.7152x over previous
import functools

import jax
import jax.numpy as jnp
from jax import lax
from jax.experimental import pallas as pl
from jax.experimental.pallas import tpu as pltpu
from jax.experimental.pallas import tpu_sc as plsc

N = 100000
E = 6400000
ROW = 128
NROWS = E // ROW
NC = 2
NS = 16
NW = NC * NS
CH = 16
CE = CH * ROW
NCHUNKS = NROWS // CH

A_FULL = N // ROW
A_TAIL_START = N - ROW


def _sc_kernel(e1d, n_id_src, n_id_dst,
               scale_src, bias_src, std_src,
               scale_dst, bias_dst, std_dst,
               out,
               idx, vbuf,
               f0, f1, f2, f3, f4, f5,
               ld0, ld1, g0, g1, st0, st1):
    c = lax.axis_index("c")
    s = lax.axis_index("s")

    fused = (f0, f1, f2, f3, f4, f5)
    sem_ld = (ld0, ld1)
    sem_g = (g0, g1)
    sem_st = (st0, st1)
    src_params = (scale_src, bias_src, std_src)
    dst_params = (scale_dst, bias_dst, std_dst)

    n_a = jnp.where(s < (A_FULL % NS), A_FULL // NS + 1, A_FULL // NS)

    def a_fire_loads(t, slot):
        start = (s + t * NS) * ROW
        pltpu.async_copy(n_id_src.at[pl.ds(start, ROW)], idx.at[slot, 0, 0], sem_ld[slot])
        pltpu.async_copy(n_id_dst.at[pl.ds(start, ROW)], idx.at[slot, 1, 0], sem_ld[slot])

    def a_wait_loads(slot):
        pltpu.make_async_copy(n_id_src.at[pl.ds(0, ROW)], idx.at[slot, 0, 0], sem_ld[slot]).wait()
        pltpu.make_async_copy(n_id_dst.at[pl.ds(0, ROW)], idx.at[slot, 1, 0], sem_ld[slot]).wait()

    def a_wait_stores(slot):
        for k in range(6):
            pltpu.make_async_copy(vbuf.at[slot, k, 0], fused[k].at[pl.ds(0, ROW)],
                                  sem_st[slot]).wait()

    def a_chunk_body(t, slot):
        start = (s + t * NS) * ROW

        @pl.when(t >= 2)
        def _():
            a_wait_stores(slot)

        @pl.when(t + 1 < n_a)
        def _():
            a_fire_loads(t + 1, 1 - slot)

        a_wait_loads(slot)
        for k in range(3):
            pltpu.async_copy(src_params[k].at[idx.at[slot, 0, 0]],
                             vbuf.at[slot, k, 0], sem_g[slot])
            pltpu.async_copy(dst_params[k].at[idx.at[slot, 1, 0]],
                             vbuf.at[slot, 3 + k, 0], sem_g[slot])
        for k in range(6):
            pltpu.make_async_copy(src_params[0].at[pl.ds(0, ROW)],
                                  vbuf.at[slot, k, 0], sem_g[slot]).wait()
        for k in range(6):
            pltpu.async_copy(vbuf.at[slot, k, 0], fused[k].at[pl.ds(start, ROW)],
                             sem_st[slot])

    a_fire_loads(0, 0)

    def body_a(p, carry):
        a_chunk_body(2 * p, 0)

        @pl.when(2 * p + 1 < n_a)
        def _():
            a_chunk_body(2 * p + 1, 1)

        return carry

    lax.fori_loop(0, (n_a + 1) // 2, body_a, 0)
    a_wait_stores(0)
    a_wait_stores(1)

    @pl.when(s == NS - 1)
    def _():
        tail_loads = [
            pltpu.async_copy(n_id_src.at[pl.ds(A_TAIL_START, ROW)], idx.at[0, 0, 0], ld0),
            pltpu.async_copy(n_id_dst.at[pl.ds(A_TAIL_START, ROW)], idx.at[0, 1, 0], ld0)]
        for cp in tail_loads:
            cp.wait()
        gs = []
        for k in range(3):
            gs.append(pltpu.async_copy(src_params[k].at[idx.at[0, 0, 0]],
                                       vbuf.at[0, k, 0], g0))
            gs.append(pltpu.async_copy(dst_params[k].at[idx.at[0, 1, 0]],
                                       vbuf.at[0, 3 + k, 0], g0))
        for g in gs:
            g.wait()
        ws = [pltpu.async_copy(vbuf.at[0, k, 0], fused[k].at[pl.ds(A_TAIL_START, ROW)], st0)
              for k in range(6)]
        for cp in ws:
            cp.wait()

    plsc.subcore_barrier()

    w = s * NC + c
    n_t = (NCHUNKS - w + NW - 1) // NW

    def fire_loads(t, slot):
        g = w + NW * t
        for j in range(CH):
            pltpu.async_copy(e1d.at[pl.ds((g * CH + j) * ROW, ROW)],
                             idx.at[slot, 0, j], sem_ld[slot])
            pltpu.async_copy(e1d.at[pl.ds(E + (g * CH + j) * ROW, ROW)],
                             idx.at[slot, 1, j], sem_ld[slot])

    def wait_loads(slot):
        for j in range(CH):
            pltpu.make_async_copy(e1d.at[pl.ds(0, ROW)], idx.at[slot, 0, j], sem_ld[slot]).wait()
            pltpu.make_async_copy(e1d.at[pl.ds(0, ROW)], idx.at[slot, 1, j], sem_ld[slot]).wait()

    def wait_block(sem, slot):
        pltpu.make_async_copy(out.at[pl.ds(0, 6), pl.ds(0, CH)], vbuf.at[slot], sem[slot]).wait()

    def chunk_body(t, slot):
        g = w + NW * t

        @pl.when(t >= 2)
        def _():
            wait_block(sem_st, slot)

        @pl.when(t + 1 < n_t)
        def _():
            fire_loads(t + 1, 1 - slot)

        wait_loads(slot)
        for j in range(CH):
            for k in range(3):
                pltpu.async_copy(fused[k].at[idx.at[slot, 0, j]],
                                 vbuf.at[slot, k, j], sem_g[slot])
                pltpu.async_copy(fused[3 + k].at[idx.at[slot, 1, j]],
                                 vbuf.at[slot, 3 + k, j], sem_g[slot])
        wait_block(sem_g, slot)
        for k in range(6):
            pltpu.async_copy(vbuf.at[slot, k], out.at[k, pl.ds(g * CH, CH)], sem_st[slot])

    fire_loads(0, 0)

    def body_b(p, carry):
        chunk_body(2 * p, 0)

        @pl.when(2 * p + 1 < n_t)
        def _():
            chunk_body(2 * p + 1, 1)

        return carry

    lax.fori_loop(0, (n_t + 1) // 2, body_b, 0)

    wait_block(sem_st, 0)
    wait_block(sem_st, 1)


@jax.jit
def _run(e1d, n_id_src, n_id_dst,
         scale_src, bias_src, std_src, scale_dst, bias_dst, std_dst):
    mesh = plsc.VectorSubcoreMesh(core_axis_name="c", subcore_axis_name="s")
    kfn = functools.partial(
        pl.kernel,
        mesh=mesh,
        out_type=jax.ShapeDtypeStruct((6, NROWS, ROW), jnp.float32),
        scratch_types=[
            pltpu.VMEM((2, 2, CH, ROW), jnp.int32),
            pltpu.VMEM((2, 6, CH, ROW), jnp.float32),
        ] + [pltpu.VMEM_SHARED((N,), jnp.float32) for _ in range(6)]
        + [pltpu.SemaphoreType.DMA for _ in range(6)],
    )(_sc_kernel)
    return kfn(e1d, n_id_src, n_id_dst,
               scale_src, bias_src, std_src,
               scale_dst, bias_dst, std_dst)


def kernel(edge_index, n_id_src, n_id_dst, scale_src, bias_src, std_src,
           scale_dst, bias_dst, std_dst):
    e1d = edge_index.reshape(2 * E)
    out = _run(e1d, n_id_src, n_id_dst,
               scale_src, bias_src, std_src,
               scale_dst, bias_dst, std_dst)
    return out.reshape(6, E)


# speedup vs baseline: 6.8490x; 1.0199x over previous
import functools

import jax
import jax.numpy as jnp
from jax import lax
from jax.experimental import pallas as pl
from jax.experimental.pallas import tpu as pltpu
from jax.experimental.pallas import tpu_sc as plsc

N = 100000
E = 6400000
ROW = 128
NROWS = E // ROW
NC = 2
NS = 16
NW = NC * NS
CH = 16
NCHUNKS = NROWS // CH

A_FULL = N // ROW
A_TAIL_START = N - ROW


def _sc_kernel(e_rows, n_id_src, n_id_dst,
               scale_src, bias_src, std_src,
               scale_dst, bias_dst, std_dst,
               out,
               idx, vbuf,
               f0, f1, f2, f3, f4, f5,
               ld0, ld1, g0, g1, st0, st1):
    c = lax.axis_index("c")
    s = lax.axis_index("s")

    fused = (f0, f1, f2, f3, f4, f5)
    sem_ld = (ld0, ld1)
    sem_g = (g0, g1)
    sem_st = (st0, st1)
    src_params = (scale_src, bias_src, std_src)
    dst_params = (scale_dst, bias_dst, std_dst)

    n_a = jnp.where(s < (A_FULL % NS), A_FULL // NS + 1, A_FULL // NS)

    def a_fire_loads(t, slot):
        start = (s + t * NS) * ROW
        pltpu.async_copy(n_id_src.at[pl.ds(start, ROW)], idx.at[slot, 0, 0], sem_ld[slot])
        pltpu.async_copy(n_id_dst.at[pl.ds(start, ROW)], idx.at[slot, 1, 0], sem_ld[slot])

    def a_wait_loads(slot):
        pltpu.make_async_copy(n_id_src.at[pl.ds(0, ROW)], idx.at[slot, 0, 0], sem_ld[slot]).wait()
        pltpu.make_async_copy(n_id_dst.at[pl.ds(0, ROW)], idx.at[slot, 1, 0], sem_ld[slot]).wait()

    def a_wait_stores(slot):
        for k in range(6):
            pltpu.make_async_copy(vbuf.at[slot, k, 0], fused[k].at[pl.ds(0, ROW)],
                                  sem_st[slot]).wait()

    def a_chunk_body(t, slot):
        start = (s + t * NS) * ROW

        @pl.when(t >= 2)
        def _():
            a_wait_stores(slot)

        @pl.when(t + 1 < n_a)
        def _():
            a_fire_loads(t + 1, 1 - slot)

        a_wait_loads(slot)
        for k in range(3):
            pltpu.async_copy(src_params[k].at[idx.at[slot, 0, 0]],
                             vbuf.at[slot, k, 0], sem_g[slot])
            pltpu.async_copy(dst_params[k].at[idx.at[slot, 1, 0]],
                             vbuf.at[slot, 3 + k, 0], sem_g[slot])
        for k in range(6):
            pltpu.make_async_copy(src_params[0].at[pl.ds(0, ROW)],
                                  vbuf.at[slot, k, 0], sem_g[slot]).wait()
        for k in range(6):
            pltpu.async_copy(vbuf.at[slot, k, 0], fused[k].at[pl.ds(start, ROW)],
                             sem_st[slot])

    a_fire_loads(0, 0)

    def body_a(p, carry):
        a_chunk_body(2 * p, 0)

        @pl.when(2 * p + 1 < n_a)
        def _():
            a_chunk_body(2 * p + 1, 1)

        return carry

    lax.fori_loop(0, (n_a + 1) // 2, body_a, 0)
    a_wait_stores(0)
    a_wait_stores(1)

    @pl.when(s == NS - 1)
    def _():
        tail_loads = [
            pltpu.async_copy(n_id_src.at[pl.ds(A_TAIL_START, ROW)], idx.at[0, 0, 0], ld0),
            pltpu.async_copy(n_id_dst.at[pl.ds(A_TAIL_START, ROW)], idx.at[0, 1, 0], ld0)]
        for cp in tail_loads:
            cp.wait()
        gs = []
        for k in range(3):
            gs.append(pltpu.async_copy(src_params[k].at[idx.at[0, 0, 0]],
                                       vbuf.at[0, k, 0], g0))
            gs.append(pltpu.async_copy(dst_params[k].at[idx.at[0, 1, 0]],
                                       vbuf.at[0, 3 + k, 0], g0))
        for g in gs:
            g.wait()
        ws = [pltpu.async_copy(vbuf.at[0, k, 0], fused[k].at[pl.ds(A_TAIL_START, ROW)], st0)
              for k in range(6)]
        for cp in ws:
            cp.wait()

    plsc.subcore_barrier()

    w = s * NC + c
    n_t = (NCHUNKS - w + NW - 1) // NW

    def fire_loads(t, slot):
        g = w + NW * t
        pltpu.async_copy(e_rows.at[0, pl.ds(g * CH, CH)], idx.at[slot, 0], sem_ld[slot])
        pltpu.async_copy(e_rows.at[1, pl.ds(g * CH, CH)], idx.at[slot, 1], sem_ld[slot])

    def wait_loads(slot):
        pltpu.make_async_copy(e_rows.at[0, pl.ds(0, CH)], idx.at[slot, 0], sem_ld[slot]).wait()
        pltpu.make_async_copy(e_rows.at[1, pl.ds(0, CH)], idx.at[slot, 1], sem_ld[slot]).wait()

    def wait_block(sem, slot):
        pltpu.make_async_copy(out.at[pl.ds(0, 6), pl.ds(0, CH)], vbuf.at[slot], sem[slot]).wait()

    def chunk_body(t, slot):
        g = w + NW * t

        @pl.when(t >= 2)
        def _():
            wait_block(sem_st, slot)

        @pl.when(t + 1 < n_t)
        def _():
            fire_loads(t + 1, 1 - slot)

        wait_loads(slot)
        for j in range(CH):
            for k in range(3):
                pltpu.async_copy(fused[k].at[idx.at[slot, 0, j]],
                                 vbuf.at[slot, k, j], sem_g[slot])
                pltpu.async_copy(fused[3 + k].at[idx.at[slot, 1, j]],
                                 vbuf.at[slot, 3 + k, j], sem_g[slot])
        wait_block(sem_g, slot)
        for k in range(6):
            pltpu.async_copy(vbuf.at[slot, k], out.at[k, pl.ds(g * CH, CH)], sem_st[slot])

    fire_loads(0, 0)

    def body_b(p, carry):
        chunk_body(2 * p, 0)

        @pl.when(2 * p + 1 < n_t)
        def _():
            chunk_body(2 * p + 1, 1)

        return carry

    lax.fori_loop(0, (n_t + 1) // 2, body_b, 0)

    wait_block(sem_st, 0)
    wait_block(sem_st, 1)


@jax.jit
def _run(e_rows, n_id_src, n_id_dst,
         scale_src, bias_src, std_src, scale_dst, bias_dst, std_dst):
    mesh = plsc.VectorSubcoreMesh(core_axis_name="c", subcore_axis_name="s")
    kfn = functools.partial(
        pl.kernel,
        mesh=mesh,
        out_type=jax.ShapeDtypeStruct((6, NROWS, ROW), jnp.float32),
        scratch_types=[
            pltpu.VMEM((2, 2, CH, ROW), jnp.int32),
            pltpu.VMEM((2, 6, CH, ROW), jnp.float32),
        ] + [pltpu.VMEM_SHARED((N,), jnp.float32) for _ in range(6)]
        + [pltpu.SemaphoreType.DMA for _ in range(6)],
    )(_sc_kernel)
    return kfn(e_rows, n_id_src, n_id_dst,
               scale_src, bias_src, std_src,
               scale_dst, bias_dst, std_dst)


def kernel(edge_index, n_id_src, n_id_dst, scale_src, bias_src, std_src,
           scale_dst, bias_dst, std_dst):
    e_rows = edge_index.reshape(2, NROWS, ROW)
    out = _run(e_rows, n_id_src, n_id_dst,
               scale_src, bias_src, std_src,
               scale_dst, bias_dst, std_dst)
    return out.reshape(6, E)
